# baseline (speedup 1.0000x reference)
import functools

import jax
import jax.numpy as jnp
from jax.experimental import pallas as pl

_B, _S, _D, _E, _DFF, _TOPK = 4, 2048, 768, 8, 128, 2
_BLK = 1024
_DH = _D // 2
_F = _E * _DFF


def _moe_block_kernel(x_ref, gW1_ref, gb1_ref, gW2_ref, gb2_ref,
                      b1cat_ref, w1cat_ref, w2cat_ref, be2s_ref,
                      lng_ref, lnb_ref,
                      out_ref, probs_ref, loss_ref):
    i = pl.program_id(0)
    nblocks = pl.num_programs(0)

    def half(xb):
        h = jnp.maximum(jnp.dot(xb, gW1_ref[...],
                                preferred_element_type=jnp.float32)
                        + gb1_ref[...], 0.0)
        a = jnp.dot(xb, w1cat_ref[...],
                    preferred_element_type=jnp.float32) + b1cat_ref[...]

        logits = jnp.dot(h, gW2_ref[...],
                         preferred_element_type=jnp.float32) + gb2_ref[...]

        eidx = jax.lax.broadcasted_iota(jnp.int32, logits.shape, 1)
        m1 = jnp.max(logits, axis=1, keepdims=True)
        i1 = jnp.min(jnp.where(logits == m1, eidx, _E), axis=1,
                     keepdims=True)
        sel1 = eidx == i1
        masked = jnp.where(sel1, -jnp.inf, logits)
        m2 = jnp.max(masked, axis=1, keepdims=True)
        i2 = jnp.min(jnp.where(masked == m2, eidx, _E), axis=1,
                     keepdims=True)

        e2 = jnp.exp(m2 - m1)
        g1 = 1.0 / (1.0 + e2)
        g2 = e2 * g1
        w = jnp.where(sel1, g1, jnp.where(eidx == i2, g2, 0.0))

        p = jnp.exp(logits - m1)
        p = p / jnp.sum(p, axis=1, keepdims=True)
        pb = jnp.sum(p, axis=0, keepdims=True)

        hgelu = a * 0.5 * (1.0 + jax.lax.erf(a * 0.7071067811865476))
        parts = [hgelu[:, e * _DFF:(e + 1) * _DFF] * w[:, e:e + 1]
                 for e in range(_E)]
        hw = jnp.concatenate(parts, axis=1)
        y = (jnp.dot(hw, w2cat_ref[...],
                     preferred_element_type=jnp.float32)
             + jnp.dot(w, be2s_ref[...],
                       preferred_element_type=jnp.float32)
             + xb)

        mu = jnp.mean(y, axis=1, keepdims=True)
        yc = y - mu
        var = jnp.mean(yc * yc, axis=1, keepdims=True)
        yn = yc * jax.lax.rsqrt(var + 1e-5) * lng_ref[...] + lnb_ref[...]
        return yn, pb

    hb = _BLK // 2
    yn0, pb0 = half(x_ref[:hb, :])
    yn1, pb1 = half(x_ref[hb:, :])
    out_ref[:hb, :] = yn0
    out_ref[hb:, :] = yn1
    pb = pb0 + pb1

    @pl.when(i == 0)
    def _():
        probs_ref[...] = pb

    @pl.when(i != 0)
    def _():
        probs_ref[...] += pb

    @pl.when(i == nblocks - 1)
    def _():
        n_tokens = nblocks * _BLK
        ep = probs_ref[...] / n_tokens
        t = 1.0 / _E
        kl = jnp.sum(t * (jnp.log(t) - jnp.log(ep + 1e-8)),
                     axis=1, keepdims=True)
        loss_ref[...] = kl


@functools.partial(jax.jit, static_argnames=())
def kernel(x, gW1, gb1, gW2, gb2, We1, be1, We2, be2, e_scale, e_bias,
           ln_g, ln_b):
    b, s, d = x.shape
    n = b * s
    xf = x.reshape(n, d)

    w1cat = jnp.transpose(We1, (1, 0, 2)).reshape(d, _F)
    b1cat = be1.reshape(1, _F)
    w2cat = (We2 * e_scale[:, None, :]).reshape(_F, d)
    be2s = be2 * e_scale + e_bias

    nblocks = n // _BLK
    grid = (nblocks,)
    out, _, loss = pl.pallas_call(
        _moe_block_kernel,
        grid=grid,
        in_specs=[
            pl.BlockSpec((_BLK, d), lambda i: (i, 0)),
            pl.BlockSpec((d, _DH), lambda i: (0, 0)),
            pl.BlockSpec((1, _DH), lambda i: (0, 0)),
            pl.BlockSpec((_DH, _E), lambda i: (0, 0)),
            pl.BlockSpec((1, _E), lambda i: (0, 0)),
            pl.BlockSpec((1, _F), lambda i: (0, 0)),
            pl.BlockSpec((d, _F), lambda i: (0, 0)),
            pl.BlockSpec((_F, d), lambda i: (0, 0)),
            pl.BlockSpec((_E, d), lambda i: (0, 0)),
            pl.BlockSpec((1, d), lambda i: (0, 0)),
            pl.BlockSpec((1, d), lambda i: (0, 0)),
        ],
        out_specs=[
            pl.BlockSpec((_BLK, d), lambda i: (i, 0)),
            pl.BlockSpec((1, _E), lambda i: (0, 0)),
            pl.BlockSpec((1, 1), lambda i: (0, 0)),
        ],
        out_shape=[
            jax.ShapeDtypeStruct((n, d), jnp.float32),
            jax.ShapeDtypeStruct((1, _E), jnp.float32),
            jax.ShapeDtypeStruct((1, 1), jnp.float32),
        ],
    )(xf, gW1, gb1.reshape(1, -1), gW2, gb2.reshape(1, -1),
      b1cat, w1cat, w2cat, be2s, ln_g.reshape(1, -1), ln_b.reshape(1, -1))

    return out.reshape(b, s, d), loss.reshape(())


# speedup vs baseline: 1.1005x; 1.1005x over previous
import functools

import jax
import jax.numpy as jnp
from jax.experimental import pallas as pl

_B, _S, _D, _E, _DFF, _TOPK = 4, 2048, 768, 8, 128, 2
_BLK = 1024
_DH = _D // 2
_F = _E * _DFF


def _moe_block_kernel(x_ref, gW1_ref, gb1_ref, gW2_ref, gb2_ref,
                      b1cat_ref, w1cat_ref, w2cat_ref, be2s_ref,
                      lng_ref, lnb_ref,
                      out_ref, probs_ref, loss_ref):
    i = pl.program_id(0)
    nblocks = pl.num_programs(0)

    def half(xb):
        h = jnp.maximum(jnp.dot(xb, gW1_ref[...],
                                preferred_element_type=jnp.float32)
                        + gb1_ref[...], 0.0)
        a = jnp.dot(xb, w1cat_ref[...],
                    preferred_element_type=jnp.float32) + b1cat_ref[...]

        logits = jnp.dot(h, gW2_ref[...],
                         preferred_element_type=jnp.float32) + gb2_ref[...]

        lt = logits.T
        eidx = jax.lax.broadcasted_iota(jnp.int32, lt.shape, 0)
        m1 = jnp.max(lt, axis=0, keepdims=True)
        i1 = jnp.min(jnp.where(lt == m1, eidx, _E), axis=0, keepdims=True)
        sel1 = eidx == i1
        masked = jnp.where(sel1, -jnp.inf, lt)
        m2 = jnp.max(masked, axis=0, keepdims=True)
        i2 = jnp.min(jnp.where(masked == m2, eidx, _E), axis=0,
                     keepdims=True)

        e2 = jnp.exp(m2 - m1)
        g1 = 1.0 / (1.0 + e2)
        g2 = e2 * g1
        wt = jnp.where(sel1, g1, jnp.where(eidx == i2, g2, 0.0))
        w = wt.T

        p = jnp.exp(lt - m1)
        p = p / jnp.sum(p, axis=0, keepdims=True)
        pb = jnp.sum(p, axis=1, keepdims=True)

        hgelu = a * 0.5 * (1.0 + jax.lax.erf(a * 0.7071067811865476))
        parts = [hgelu[:, e * _DFF:(e + 1) * _DFF] * w[:, e:e + 1]
                 for e in range(_E)]
        hw = jnp.concatenate(parts, axis=1)
        y = (jnp.dot(hw, w2cat_ref[...],
                     preferred_element_type=jnp.float32)
             + jnp.dot(w, be2s_ref[...],
                       preferred_element_type=jnp.float32)
             + xb)

        mu = jnp.mean(y, axis=1, keepdims=True)
        yc = y - mu
        var = jnp.mean(yc * yc, axis=1, keepdims=True)
        yn = yc * jax.lax.rsqrt(var + 1e-5) * lng_ref[...] + lnb_ref[...]
        return yn, pb

    hb = _BLK // 2
    yn0, pb0 = half(x_ref[:hb, :])
    yn1, pb1 = half(x_ref[hb:, :])
    out_ref[:hb, :] = yn0
    out_ref[hb:, :] = yn1
    pb = pb0 + pb1

    @pl.when(i == 0)
    def _():
        probs_ref[...] = pb

    @pl.when(i != 0)
    def _():
        probs_ref[...] += pb

    @pl.when(i == nblocks - 1)
    def _():
        n_tokens = nblocks * _BLK
        ep = probs_ref[...] / n_tokens
        t = 1.0 / _E
        kl = jnp.sum(t * (jnp.log(t) - jnp.log(ep + 1e-8)),
                     axis=0, keepdims=True)
        loss_ref[...] = kl


@functools.partial(jax.jit, static_argnames=())
def kernel(x, gW1, gb1, gW2, gb2, We1, be1, We2, be2, e_scale, e_bias,
           ln_g, ln_b):
    b, s, d = x.shape
    n = b * s
    xf = x.reshape(n, d)

    w1cat = jnp.transpose(We1, (1, 0, 2)).reshape(d, _F)
    b1cat = be1.reshape(1, _F)
    w2cat = (We2 * e_scale[:, None, :]).reshape(_F, d)
    be2s = be2 * e_scale + e_bias

    nblocks = n // _BLK
    grid = (nblocks,)
    out, _, loss = pl.pallas_call(
        _moe_block_kernel,
        grid=grid,
        in_specs=[
            pl.BlockSpec((_BLK, d), lambda i: (i, 0)),
            pl.BlockSpec((d, _DH), lambda i: (0, 0)),
            pl.BlockSpec((1, _DH), lambda i: (0, 0)),
            pl.BlockSpec((_DH, _E), lambda i: (0, 0)),
            pl.BlockSpec((1, _E), lambda i: (0, 0)),
            pl.BlockSpec((1, _F), lambda i: (0, 0)),
            pl.BlockSpec((d, _F), lambda i: (0, 0)),
            pl.BlockSpec((_F, d), lambda i: (0, 0)),
            pl.BlockSpec((_E, d), lambda i: (0, 0)),
            pl.BlockSpec((1, d), lambda i: (0, 0)),
            pl.BlockSpec((1, d), lambda i: (0, 0)),
        ],
        out_specs=[
            pl.BlockSpec((_BLK, d), lambda i: (i, 0)),
            pl.BlockSpec((_E, 1), lambda i: (0, 0)),
            pl.BlockSpec((1, 1), lambda i: (0, 0)),
        ],
        out_shape=[
            jax.ShapeDtypeStruct((n, d), jnp.float32),
            jax.ShapeDtypeStruct((_E, 1), jnp.float32),
            jax.ShapeDtypeStruct((1, 1), jnp.float32),
        ],
    )(xf, gW1, gb1.reshape(1, -1), gW2, gb2.reshape(1, -1),
      b1cat, w1cat, w2cat, be2s, ln_g.reshape(1, -1), ln_b.reshape(1, -1))

    return out.reshape(b, s, d), loss.reshape(())


# speedup vs baseline: 1.1299x; 1.0267x over previous
import functools

import jax
import jax.numpy as jnp
from jax.experimental import pallas as pl

_B, _S, _D, _E, _DFF, _TOPK = 4, 2048, 768, 8, 128, 2
_BLK = 2048
_HB = 512
_DH = _D // 2
_F = _E * _DFF


def _moe_block_kernel(x_ref, gW1_ref, gb1_ref, gW2_ref, gb2_ref,
                      b1cat_ref, w1cat_ref, w2cat_ref, be2s_ref,
                      lng_ref, lnb_ref,
                      out_ref, probs_ref, loss_ref):
    i = pl.program_id(0)
    nblocks = pl.num_programs(0)

    def half(xb):
        h = jnp.maximum(jnp.dot(xb, gW1_ref[...],
                                preferred_element_type=jnp.float32)
                        + gb1_ref[...], 0.0)
        a = jnp.dot(xb, w1cat_ref[...],
                    preferred_element_type=jnp.float32) + b1cat_ref[...]

        logits = jnp.dot(h, gW2_ref[...],
                         preferred_element_type=jnp.float32) + gb2_ref[...]

        lt = logits.T
        eidx = jax.lax.broadcasted_iota(jnp.int32, lt.shape, 0)
        m1 = jnp.max(lt, axis=0, keepdims=True)
        i1 = jnp.min(jnp.where(lt == m1, eidx, _E), axis=0, keepdims=True)
        sel1 = eidx == i1
        masked = jnp.where(sel1, -jnp.inf, lt)
        m2 = jnp.max(masked, axis=0, keepdims=True)
        i2 = jnp.min(jnp.where(masked == m2, eidx, _E), axis=0,
                     keepdims=True)

        e2 = jnp.exp(m2 - m1)
        g1 = 1.0 / (1.0 + e2)
        g2 = e2 * g1
        wt = jnp.where(sel1, g1, jnp.where(eidx == i2, g2, 0.0))
        w = wt.T

        p = jnp.exp(lt - m1)
        p = p / jnp.sum(p, axis=0, keepdims=True)
        pb = jnp.sum(p, axis=1, keepdims=True)

        hgelu = a * 0.5 * (1.0 + jax.lax.erf(a * 0.7071067811865476))
        parts = [hgelu[:, e * _DFF:(e + 1) * _DFF] * w[:, e:e + 1]
                 for e in range(_E)]
        hw = jnp.concatenate(parts, axis=1)
        y = (jnp.dot(hw, w2cat_ref[...],
                     preferred_element_type=jnp.float32)
             + jnp.dot(w, be2s_ref[...],
                       preferred_element_type=jnp.float32)
             + xb)

        mu = jnp.mean(y, axis=1, keepdims=True)
        yc = y - mu
        var = jnp.mean(yc * yc, axis=1, keepdims=True)
        yn = yc * jax.lax.rsqrt(var + 1e-5) * lng_ref[...] + lnb_ref[...]
        return yn, pb

    pb = None
    for q in range(_BLK // _HB):
        yn_q, pb_q = half(x_ref[q * _HB:(q + 1) * _HB, :])
        out_ref[q * _HB:(q + 1) * _HB, :] = yn_q
        pb = pb_q if pb is None else pb + pb_q

    @pl.when(i == 0)
    def _():
        probs_ref[...] = pb

    @pl.when(i != 0)
    def _():
        probs_ref[...] += pb

    @pl.when(i == nblocks - 1)
    def _():
        n_tokens = nblocks * _BLK
        ep = probs_ref[...] / n_tokens
        t = 1.0 / _E
        kl = jnp.sum(t * (jnp.log(t) - jnp.log(ep + 1e-8)),
                     axis=0, keepdims=True)
        loss_ref[...] = kl


@functools.partial(jax.jit, static_argnames=())
def kernel(x, gW1, gb1, gW2, gb2, We1, be1, We2, be2, e_scale, e_bias,
           ln_g, ln_b):
    b, s, d = x.shape
    n = b * s
    xf = x.reshape(n, d)

    w1cat = jnp.transpose(We1, (1, 0, 2)).reshape(d, _F)
    b1cat = be1.reshape(1, _F)
    w2cat = (We2 * e_scale[:, None, :]).reshape(_F, d)
    be2s = be2 * e_scale + e_bias

    nblocks = n // _BLK
    grid = (nblocks,)
    out, _, loss = pl.pallas_call(
        _moe_block_kernel,
        grid=grid,
        in_specs=[
            pl.BlockSpec((_BLK, d), lambda i: (i, 0)),
            pl.BlockSpec((d, _DH), lambda i: (0, 0)),
            pl.BlockSpec((1, _DH), lambda i: (0, 0)),
            pl.BlockSpec((_DH, _E), lambda i: (0, 0)),
            pl.BlockSpec((1, _E), lambda i: (0, 0)),
            pl.BlockSpec((1, _F), lambda i: (0, 0)),
            pl.BlockSpec((d, _F), lambda i: (0, 0)),
            pl.BlockSpec((_F, d), lambda i: (0, 0)),
            pl.BlockSpec((_E, d), lambda i: (0, 0)),
            pl.BlockSpec((1, d), lambda i: (0, 0)),
            pl.BlockSpec((1, d), lambda i: (0, 0)),
        ],
        out_specs=[
            pl.BlockSpec((_BLK, d), lambda i: (i, 0)),
            pl.BlockSpec((_E, 1), lambda i: (0, 0)),
            pl.BlockSpec((1, 1), lambda i: (0, 0)),
        ],
        out_shape=[
            jax.ShapeDtypeStruct((n, d), jnp.float32),
            jax.ShapeDtypeStruct((_E, 1), jnp.float32),
            jax.ShapeDtypeStruct((1, 1), jnp.float32),
        ],
    )(xf, gW1, gb1.reshape(1, -1), gW2, gb2.reshape(1, -1),
      b1cat, w1cat, w2cat, be2s, ln_g.reshape(1, -1), ln_b.reshape(1, -1))

    return out.reshape(b, s, d), loss.reshape(())


# speedup vs baseline: 1.1346x; 1.0041x over previous
import functools

import jax
import jax.numpy as jnp
from jax.experimental import pallas as pl

_B, _S, _D, _E, _DFF, _TOPK = 4, 2048, 768, 8, 128, 2
_BLK = 2048
_HB = 512
_DH = _D // 2
_F = _E * _DFF


def _moe_block_kernel(x_ref, gW1_ref, gb1_ref, gW2_ref, gb2_ref,
                      b1cat_ref, w1cat_ref, w2cat_ref, be2s_ref,
                      lng_ref, lnb_ref,
                      out_ref, probs_ref, loss_ref):
    i = pl.program_id(0)
    nblocks = pl.num_programs(0)

    def half(xb):
        h = jnp.maximum(jnp.dot(xb, gW1_ref[...],
                                preferred_element_type=jnp.float32)
                        + gb1_ref[...], 0.0)
        a = jnp.dot(xb.astype(jnp.bfloat16), w1cat_ref[...],
                    preferred_element_type=jnp.float32) + b1cat_ref[...]

        logits = jnp.dot(h, gW2_ref[...],
                         preferred_element_type=jnp.float32) + gb2_ref[...]

        lt = logits.T
        eidx = jax.lax.broadcasted_iota(jnp.int32, lt.shape, 0)
        m1 = jnp.max(lt, axis=0, keepdims=True)
        i1 = jnp.min(jnp.where(lt == m1, eidx, _E), axis=0, keepdims=True)
        sel1 = eidx == i1
        masked = jnp.where(sel1, -jnp.inf, lt)
        m2 = jnp.max(masked, axis=0, keepdims=True)
        i2 = jnp.min(jnp.where(masked == m2, eidx, _E), axis=0,
                     keepdims=True)

        e2 = jnp.exp(m2 - m1)
        g1 = 1.0 / (1.0 + e2)
        g2 = e2 * g1
        wt = jnp.where(sel1, g1, jnp.where(eidx == i2, g2, 0.0))
        w = wt.T

        p = jnp.exp(lt - m1)
        p = p / jnp.sum(p, axis=0, keepdims=True)
        pb = jnp.sum(p, axis=1, keepdims=True)

        hgelu = a * 0.5 * (1.0 + jax.lax.erf(a * 0.7071067811865476))
        parts = [hgelu[:, e * _DFF:(e + 1) * _DFF] * w[:, e:e + 1]
                 for e in range(_E)]
        hw = jnp.concatenate(parts, axis=1).astype(jnp.bfloat16)
        y = (jnp.dot(hw, w2cat_ref[...],
                     preferred_element_type=jnp.float32)
             + jnp.dot(w, be2s_ref[...],
                       preferred_element_type=jnp.float32)
             + xb)

        mu = jnp.mean(y, axis=1, keepdims=True)
        yc = y - mu
        var = jnp.mean(yc * yc, axis=1, keepdims=True)
        yn = yc * jax.lax.rsqrt(var + 1e-5) * lng_ref[...] + lnb_ref[...]
        return yn, pb

    pb = None
    for q in range(_BLK // _HB):
        yn_q, pb_q = half(x_ref[q * _HB:(q + 1) * _HB, :])
        out_ref[q * _HB:(q + 1) * _HB, :] = yn_q
        pb = pb_q if pb is None else pb + pb_q

    @pl.when(i == 0)
    def _():
        probs_ref[...] = pb

    @pl.when(i != 0)
    def _():
        probs_ref[...] += pb

    @pl.when(i == nblocks - 1)
    def _():
        n_tokens = nblocks * _BLK
        ep = probs_ref[...] / n_tokens
        t = 1.0 / _E
        kl = jnp.sum(t * (jnp.log(t) - jnp.log(ep + 1e-8)),
                     axis=0, keepdims=True)
        loss_ref[...] = kl


@functools.partial(jax.jit, static_argnames=())
def kernel(x, gW1, gb1, gW2, gb2, We1, be1, We2, be2, e_scale, e_bias,
           ln_g, ln_b):
    b, s, d = x.shape
    n = b * s
    xf = x.reshape(n, d)

    w1cat = jnp.transpose(We1, (1, 0, 2)).reshape(d, _F).astype(jnp.bfloat16)
    b1cat = be1.reshape(1, _F)
    w2cat = (We2 * e_scale[:, None, :]).reshape(_F, d).astype(jnp.bfloat16)
    be2s = be2 * e_scale + e_bias

    nblocks = n // _BLK
    grid = (nblocks,)
    out, _, loss = pl.pallas_call(
        _moe_block_kernel,
        grid=grid,
        in_specs=[
            pl.BlockSpec((_BLK, d), lambda i: (i, 0)),
            pl.BlockSpec((d, _DH), lambda i: (0, 0)),
            pl.BlockSpec((1, _DH), lambda i: (0, 0)),
            pl.BlockSpec((_DH, _E), lambda i: (0, 0)),
            pl.BlockSpec((1, _E), lambda i: (0, 0)),
            pl.BlockSpec((1, _F), lambda i: (0, 0)),
            pl.BlockSpec((d, _F), lambda i: (0, 0)),
            pl.BlockSpec((_F, d), lambda i: (0, 0)),
            pl.BlockSpec((_E, d), lambda i: (0, 0)),
            pl.BlockSpec((1, d), lambda i: (0, 0)),
            pl.BlockSpec((1, d), lambda i: (0, 0)),
        ],
        out_specs=[
            pl.BlockSpec((_BLK, d), lambda i: (i, 0)),
            pl.BlockSpec((_E, 1), lambda i: (0, 0)),
            pl.BlockSpec((1, 1), lambda i: (0, 0)),
        ],
        out_shape=[
            jax.ShapeDtypeStruct((n, d), jnp.float32),
            jax.ShapeDtypeStruct((_E, 1), jnp.float32),
            jax.ShapeDtypeStruct((1, 1), jnp.float32),
        ],
    )(xf, gW1, gb1.reshape(1, -1), gW2, gb2.reshape(1, -1),
      b1cat, w1cat, w2cat, be2s, ln_g.reshape(1, -1), ln_b.reshape(1, -1))

    return out.reshape(b, s, d), loss.reshape(())


# speedup vs baseline: 1.1696x; 1.0308x over previous
import functools

import jax
import jax.numpy as jnp
from jax.experimental import pallas as pl

_B, _S, _D, _E, _DFF, _TOPK = 4, 2048, 768, 8, 128, 2
_BLK = 2048
_HB = 512
_DH = _D // 2
_F = _E * _DFF


def _moe_block_kernel(x_ref, gW1_ref, gb1_ref, gW2_ref, gb2_ref,
                      b1cat_ref, w1cat_ref, w2cat_ref, be2s_ref,
                      lng_ref, lnb_ref,
                      out_ref, probs_ref, loss_ref):
    i = pl.program_id(0)
    nblocks = pl.num_programs(0)

    def half(xb):
        h = jnp.maximum(jnp.dot(xb, gW1_ref[...],
                                preferred_element_type=jnp.float32)
                        + gb1_ref[...], 0.0)
        a = jnp.dot(xb.astype(jnp.bfloat16), w1cat_ref[...],
                    preferred_element_type=jnp.float32) + b1cat_ref[...]

        logits = jnp.dot(h, gW2_ref[...],
                         preferred_element_type=jnp.float32) + gb2_ref[...]

        lt = logits.T
        eidx = jax.lax.broadcasted_iota(jnp.int32, lt.shape, 0)
        m1 = jnp.max(lt, axis=0, keepdims=True)
        i1 = jnp.min(jnp.where(lt == m1, eidx, _E), axis=0, keepdims=True)
        sel1 = eidx == i1
        masked = jnp.where(sel1, -jnp.inf, lt)
        m2 = jnp.max(masked, axis=0, keepdims=True)
        i2 = jnp.min(jnp.where(masked == m2, eidx, _E), axis=0,
                     keepdims=True)

        e2 = jnp.exp(m2 - m1)
        g1 = 1.0 / (1.0 + e2)
        g2 = e2 * g1
        wt = jnp.where(sel1, g1, jnp.where(eidx == i2, g2, 0.0))
        w = wt.T

        p = jnp.exp(lt - m1)
        p = p / jnp.sum(p, axis=0, keepdims=True)
        pb = jnp.sum(p, axis=1, keepdims=True)

        hgelu = (a * 0.5 * (1.0 + jax.lax.erf(a * 0.7071067811865476))
                 ).astype(jnp.bfloat16)
        wb = w.astype(jnp.bfloat16)
        parts = [hgelu[:, e * _DFF:(e + 1) * _DFF] * wb[:, e:e + 1]
                 for e in range(_E)]
        hw = jnp.concatenate(parts, axis=1)
        y = (jnp.dot(hw, w2cat_ref[...],
                     preferred_element_type=jnp.float32)
             + jnp.dot(w, be2s_ref[...],
                       preferred_element_type=jnp.float32)
             + xb)

        mu = jnp.mean(y, axis=1, keepdims=True)
        yc = y - mu
        var = jnp.mean(yc * yc, axis=1, keepdims=True)
        yn = yc * jax.lax.rsqrt(var + 1e-5) * lng_ref[...] + lnb_ref[...]
        return yn, pb

    pb = None
    for q in range(_BLK // _HB):
        yn_q, pb_q = half(x_ref[q * _HB:(q + 1) * _HB, :])
        out_ref[q * _HB:(q + 1) * _HB, :] = yn_q
        pb = pb_q if pb is None else pb + pb_q

    @pl.when(i == 0)
    def _():
        probs_ref[...] = pb

    @pl.when(i != 0)
    def _():
        probs_ref[...] += pb

    @pl.when(i == nblocks - 1)
    def _():
        n_tokens = nblocks * _BLK
        ep = probs_ref[...] / n_tokens
        t = 1.0 / _E
        kl = jnp.sum(t * (jnp.log(t) - jnp.log(ep + 1e-8)),
                     axis=0, keepdims=True)
        loss_ref[...] = kl


@functools.partial(jax.jit, static_argnames=())
def kernel(x, gW1, gb1, gW2, gb2, We1, be1, We2, be2, e_scale, e_bias,
           ln_g, ln_b):
    b, s, d = x.shape
    n = b * s
    xf = x.reshape(n, d)

    w1cat = jnp.transpose(We1, (1, 0, 2)).reshape(d, _F).astype(jnp.bfloat16)
    b1cat = be1.reshape(1, _F)
    w2cat = (We2 * e_scale[:, None, :]).reshape(_F, d).astype(jnp.bfloat16)
    be2s = be2 * e_scale + e_bias

    nblocks = n // _BLK
    grid = (nblocks,)
    out, _, loss = pl.pallas_call(
        _moe_block_kernel,
        grid=grid,
        in_specs=[
            pl.BlockSpec((_BLK, d), lambda i: (i, 0)),
            pl.BlockSpec((d, _DH), lambda i: (0, 0)),
            pl.BlockSpec((1, _DH), lambda i: (0, 0)),
            pl.BlockSpec((_DH, _E), lambda i: (0, 0)),
            pl.BlockSpec((1, _E), lambda i: (0, 0)),
            pl.BlockSpec((1, _F), lambda i: (0, 0)),
            pl.BlockSpec((d, _F), lambda i: (0, 0)),
            pl.BlockSpec((_F, d), lambda i: (0, 0)),
            pl.BlockSpec((_E, d), lambda i: (0, 0)),
            pl.BlockSpec((1, d), lambda i: (0, 0)),
            pl.BlockSpec((1, d), lambda i: (0, 0)),
        ],
        out_specs=[
            pl.BlockSpec((_BLK, d), lambda i: (i, 0)),
            pl.BlockSpec((_E, 1), lambda i: (0, 0)),
            pl.BlockSpec((1, 1), lambda i: (0, 0)),
        ],
        out_shape=[
            jax.ShapeDtypeStruct((n, d), jnp.float32),
            jax.ShapeDtypeStruct((_E, 1), jnp.float32),
            jax.ShapeDtypeStruct((1, 1), jnp.float32),
        ],
    )(xf, gW1, gb1.reshape(1, -1), gW2, gb2.reshape(1, -1),
      b1cat, w1cat, w2cat, be2s, ln_g.reshape(1, -1), ln_b.reshape(1, -1))

    return out.reshape(b, s, d), loss.reshape(())
